# baseline (speedup 1.0000x reference)
import functools

import jax
import jax.numpy as jnp
from jax import lax
from jax.experimental import pallas as pl
from jax.experimental.pallas import tpu as pltpu
from jax.experimental.pallas import tpu_sc as plsc

_N = 10000
_E = 320000
_D = 128
_NP = 10240
_EP = 327680
_PAD_BIN = 10016
_IB = 256
_JC = 1024

def _hist_sc_body(src_hbm, hist_hbm, idx_v, ones_v, zeros_v, hist_sh):
    c = lax.axis_index("c")
    s = lax.axis_index("s")
    w = c * 16 + s
    for k in range(8):
        ones_v[pl.ds(k * 16, 16)] = jnp.full((16,), 1.0, jnp.float32)
    for k in range(40):
        zeros_v[pl.ds(k * 16, 16)] = jnp.zeros((16,), jnp.float32)
    pltpu.sync_copy(zeros_v, hist_sh.at[pl.ds(s * 640, 640)])
    pltpu.sync_copy(src_hbm.at[w], idx_v)
    plsc.subcore_barrier()

    def chunk(j, carry):
        pltpu.sync_copy(ones_v, hist_sh.at[idx_v.at[j]], add=True)
        return carry

    lax.fori_loop(0, 80, chunk, 0)
    plsc.subcore_barrier()
    pltpu.sync_copy(hist_sh.at[pl.ds(s * 640, 640)],
                    hist_hbm.at[c, pl.ds(s * 640, 640)])


def _rank_body(hist_a, hist_b, noise_r, histT, noise_c, out_ref):
    g = pl.program_id(0)
    ki = histT[:, 0:1] + histT[:, 1:2] + noise_c[...]
    ii = g * _IB + lax.broadcasted_iota(jnp.int32, (_IB, 1), 0)

    def kj(jc):
        return (hist_a[pl.ds(jc, 1), :] + hist_b[pl.ds(jc, 1), :]
                + noise_r[pl.ds(jc, 1), :])

    def le_body(jc, acc):
        return acc + jnp.where(kj(jc) <= ki, 1.0, 0.0)

    def lt_body(jc, acc):
        return acc + jnp.where(kj(jc) < ki, 1.0, 0.0)

    def mid_body(jc, acc):
        k = kj(jc)
        jj = jc * 128 + lax.broadcasted_iota(jnp.int32, (1, 128), 1)
        before = (k < ki) | ((k == ki) & (jj < ii))
        return acc + jnp.where(before, 1.0, 0.0)

    acc = jnp.zeros((_IB, 128), jnp.float32)
    acc = lax.fori_loop(0, 2 * g, le_body, acc)
    acc = mid_body(2 * g, acc)
    acc = mid_body(2 * g + 1, acc)
    acc = lax.fori_loop(2 * g + 2, _NP // 128, lt_body, acc)
    out_ref[...] = jnp.sum(acc, axis=1, keepdims=True).astype(jnp.int32)


_rank_tc = pl.pallas_call(
    _rank_body,
    grid=(_NP // _IB,),
    in_specs=[
        pl.BlockSpec((_NP // 128, 128), lambda g: (0, 0)),
        pl.BlockSpec((_NP // 128, 128), lambda g: (0, 0)),
        pl.BlockSpec((_NP // 128, 128), lambda g: (0, 0)),
        pl.BlockSpec((_IB, 2), lambda g: (g, 0)),
        pl.BlockSpec((_IB, 1), lambda g: (g, 0)),
    ],
    out_specs=pl.BlockSpec((_IB, 1), lambda g: (g, 0)),
    out_shape=jax.ShapeDtypeStruct((_NP, 1), jnp.int32),
)


def _permute_sc_body(x_hbm, rankf_hbm, xs_hbm, idx_hbm,
                     rk_v, rows_v, vals_v, trk_v, trows_v, tvals_v):
    c = lax.axis_index("c")
    s = lax.axis_index("s")
    w = c * 16 + s

    def do_chunk(ci):
        pltpu.sync_copy(rankf_hbm.at[pl.ds(ci * 128, 128)], rk_v)
        pltpu.sync_copy(x_hbm.at[pl.ds(ci * 128, 128), :], rows_v)
        for k in range(8):
            vals_v[pl.ds(k * 16, 16)] = (
                ci * 128 + k * 16 + lax.broadcasted_iota(jnp.int32, (16,), 0))
        pltpu.sync_copy(rows_v, xs_hbm.at[rk_v])
        pltpu.sync_copy(vals_v, idx_hbm.at[rk_v])

    do_chunk(w)
    do_chunk(w + 32)

    @pl.when(w < 14)
    def _():
        do_chunk(w + 64)

    @pl.when(w == 14)
    def _():
        pltpu.sync_copy(rankf_hbm.at[pl.ds(9984, 16)], trk_v)
        pltpu.sync_copy(x_hbm.at[pl.ds(9984, 16), :], trows_v)
        tvals_v[...] = 9984 + lax.broadcasted_iota(jnp.int32, (16,), 0)
        pltpu.sync_copy(trows_v, xs_hbm.at[trk_v])
        pltpu.sync_copy(tvals_v, idx_hbm.at[trk_v])


@functools.lru_cache(maxsize=1)
def _sc_kernels():
    mesh = plsc.VectorSubcoreMesh(core_axis_name="c", subcore_axis_name="s",
                                  num_cores=2, num_subcores=16)
    hist_sc = pl.kernel(
        _hist_sc_body,
        out_type=jax.ShapeDtypeStruct((2, _NP), jnp.float32),
        mesh=mesh,
        scratch_types=[
            pltpu.VMEM((80, 128), jnp.int32),
            pltpu.VMEM((128,), jnp.float32),
            pltpu.VMEM((640,), jnp.float32),
            pltpu.VMEM_SHARED((_NP,), jnp.float32),
        ],
    )
    permute_sc = pl.kernel(
        _permute_sc_body,
        out_type=(jax.ShapeDtypeStruct((_N, _D), jnp.float32),
                  jax.ShapeDtypeStruct((_N,), jnp.int32)),
        mesh=mesh,
        scratch_types=[
            pltpu.VMEM((128,), jnp.int32),
            pltpu.VMEM((128, _D), jnp.float32),
            pltpu.VMEM((128,), jnp.int32),
            pltpu.VMEM((16,), jnp.int32),
            pltpu.VMEM((16, _D), jnp.float32),
            pltpu.VMEM((16,), jnp.int32),
        ],
    )
    return hist_sc, permute_sc


def kernel(x, edge_index):
    hist_sc, permute_sc = _sc_kernels()
    x2 = x if x.ndim == 2 else x[0]
    src = edge_index[0].astype(jnp.int32)
    src_pad = jnp.concatenate(
        [src, jnp.full((_EP - _E,), _PAD_BIN, jnp.int32)]).reshape(32, 80, 128)

    noise = (jax.random.uniform(jax.random.key(1), (1, _N), dtype=jnp.float32)
             * 0.1)
    noise_pad = jnp.concatenate(
        [noise[0], jnp.full((_NP - _N,), jnp.inf, jnp.float32)])

    hist2 = hist_sc(src_pad)
    hist_a = hist2[0].reshape(_NP // 128, 128)
    hist_b = hist2[1].reshape(_NP // 128, 128)
    noise_r = noise_pad.reshape(_NP // 128, 128)
    histT = hist2.T
    noise_c = noise_pad.reshape(_NP, 1)

    ranks = _rank_tc(hist_a, hist_b, noise_r, histT, noise_c)
    rankf = ranks.reshape(_NP)

    xs, sidx = permute_sc(x2, rankf)
    return (xs[None], sidx[None])


# speedup vs baseline: 1.7687x; 1.7687x over previous
import functools

import jax
import jax.numpy as jnp
from jax import lax
from jax.experimental import pallas as pl
from jax.experimental.pallas import tpu as pltpu
from jax.experimental.pallas import tpu_sc as plsc

_N = 10000
_E = 320000
_D = 128
_NP = 10240
_EP = 327680
_PAD_BIN = 10016
_IB = 256
_JC = 1024

def _hist_sc_body(src_hbm, hist_hbm, idx_v, ones_v, zeros_v, hist_sh):
    c = lax.axis_index("c")
    s = lax.axis_index("s")
    w = c * 16 + s
    for k in range(8):
        ones_v[pl.ds(k * 16, 16)] = jnp.full((16,), 1.0, jnp.float32)
    for k in range(40):
        zeros_v[pl.ds(k * 16, 16)] = jnp.zeros((16,), jnp.float32)
    pltpu.sync_copy(zeros_v, hist_sh.at[pl.ds(s * 640, 640)])
    pltpu.sync_copy(src_hbm.at[w], idx_v)
    plsc.subcore_barrier()

    def chunk(j, carry):
        pltpu.sync_copy(ones_v, hist_sh.at[idx_v.at[j]], add=True)
        return carry

    lax.fori_loop(0, 80, chunk, 0)
    plsc.subcore_barrier()
    pltpu.sync_copy(hist_sh.at[pl.ds(s * 640, 640)],
                    hist_hbm.at[c, pl.ds(s * 640, 640)])


def _rank_body(hist_a, hist_b, noise_r, histT, noise_c, out_ref):
    g = pl.program_id(0)
    ki = histT[:, 0:1] + histT[:, 1:2] + noise_c[...]
    ii = g * _IB + lax.broadcasted_iota(jnp.int32, (_IB, 1), 0)

    def kj(jc):
        return (hist_a[pl.ds(jc, 1), :] + hist_b[pl.ds(jc, 1), :]
                + noise_r[pl.ds(jc, 1), :])

    def reduce8(cnt, acc):
        for k in range(_JC // 128):
            acc = acc + cnt[:, k * 128:(k + 1) * 128]
        return acc

    def le_body(jc, acc):
        return reduce8(jnp.where(kj(jc) <= ki, 1.0, 0.0), acc)

    def lt_body(jc, acc):
        return reduce8(jnp.where(kj(jc) < ki, 1.0, 0.0), acc)

    def mid_body(jc, acc):
        k = kj(jc)
        jj = jc * _JC + lax.broadcasted_iota(jnp.int32, (1, _JC), 1)
        before = (k < ki) | ((k == ki) & (jj < ii))
        return reduce8(jnp.where(before, 1.0, 0.0), acc)

    gd = g // (_JC // _IB)
    acc = jnp.zeros((_IB, 128), jnp.float32)
    acc = lax.fori_loop(0, gd, le_body, acc)
    acc = mid_body(gd, acc)
    acc = lax.fori_loop(gd + 1, _NP // _JC, lt_body, acc)
    out_ref[...] = jnp.sum(acc, axis=1, keepdims=True).astype(jnp.int32)


_rank_tc = pl.pallas_call(
    _rank_body,
    grid=(_NP // _IB,),
    in_specs=[
        pl.BlockSpec((_NP // _JC, _JC), lambda g: (0, 0)),
        pl.BlockSpec((_NP // _JC, _JC), lambda g: (0, 0)),
        pl.BlockSpec((_NP // _JC, _JC), lambda g: (0, 0)),
        pl.BlockSpec((_IB, 2), lambda g: (g, 0)),
        pl.BlockSpec((_IB, 1), lambda g: (g, 0)),
    ],
    out_specs=pl.BlockSpec((_IB, 1), lambda g: (g, 0)),
    out_shape=jax.ShapeDtypeStruct((_NP, 1), jnp.int32),
)


def _permute_sc_body(x_hbm, rankf_hbm, xs_hbm, idx_hbm,
                     rk_v, rows_v, vals_v, trk_v, trows_v, tvals_v):
    c = lax.axis_index("c")
    s = lax.axis_index("s")
    w = c * 16 + s

    def do_chunk(ci):
        pltpu.sync_copy(rankf_hbm.at[pl.ds(ci * 128, 128)], rk_v)
        pltpu.sync_copy(x_hbm.at[pl.ds(ci * 128, 128), :], rows_v)
        for k in range(8):
            vals_v[pl.ds(k * 16, 16)] = (
                ci * 128 + k * 16 + lax.broadcasted_iota(jnp.int32, (16,), 0))
        pltpu.sync_copy(rows_v, xs_hbm.at[rk_v])
        pltpu.sync_copy(vals_v, idx_hbm.at[rk_v])

    do_chunk(w)
    do_chunk(w + 32)

    @pl.when(w < 14)
    def _():
        do_chunk(w + 64)

    @pl.when(w == 14)
    def _():
        pltpu.sync_copy(rankf_hbm.at[pl.ds(9984, 16)], trk_v)
        pltpu.sync_copy(x_hbm.at[pl.ds(9984, 16), :], trows_v)
        tvals_v[...] = 9984 + lax.broadcasted_iota(jnp.int32, (16,), 0)
        pltpu.sync_copy(trows_v, xs_hbm.at[trk_v])
        pltpu.sync_copy(tvals_v, idx_hbm.at[trk_v])


@functools.lru_cache(maxsize=1)
def _sc_kernels():
    mesh = plsc.VectorSubcoreMesh(core_axis_name="c", subcore_axis_name="s",
                                  num_cores=2, num_subcores=16)
    hist_sc = pl.kernel(
        _hist_sc_body,
        out_type=jax.ShapeDtypeStruct((2, _NP), jnp.float32),
        mesh=mesh,
        scratch_types=[
            pltpu.VMEM((80, 128), jnp.int32),
            pltpu.VMEM((128,), jnp.float32),
            pltpu.VMEM((640,), jnp.float32),
            pltpu.VMEM_SHARED((_NP,), jnp.float32),
        ],
    )
    permute_sc = pl.kernel(
        _permute_sc_body,
        out_type=(jax.ShapeDtypeStruct((_N, _D), jnp.float32),
                  jax.ShapeDtypeStruct((_N,), jnp.int32)),
        mesh=mesh,
        scratch_types=[
            pltpu.VMEM((128,), jnp.int32),
            pltpu.VMEM((128, _D), jnp.float32),
            pltpu.VMEM((128,), jnp.int32),
            pltpu.VMEM((16,), jnp.int32),
            pltpu.VMEM((16, _D), jnp.float32),
            pltpu.VMEM((16,), jnp.int32),
        ],
    )
    return hist_sc, permute_sc


def kernel(x, edge_index):
    hist_sc, permute_sc = _sc_kernels()
    x2 = x if x.ndim == 2 else x[0]
    src = edge_index[0].astype(jnp.int32)
    src_pad = jnp.concatenate(
        [src, jnp.full((_EP - _E,), _PAD_BIN, jnp.int32)]).reshape(32, 80, 128)

    noise = (jax.random.uniform(jax.random.key(1), (1, _N), dtype=jnp.float32)
             * 0.1)
    noise_pad = jnp.concatenate(
        [noise[0], jnp.full((_NP - _N,), jnp.inf, jnp.float32)])

    hist2 = hist_sc(src_pad)
    hist_a = hist2[0].reshape(_NP // _JC, _JC)
    hist_b = hist2[1].reshape(_NP // _JC, _JC)
    noise_r = noise_pad.reshape(_NP // _JC, _JC)
    histT = hist2.T
    noise_c = noise_pad.reshape(_NP, 1)

    ranks = _rank_tc(hist_a, hist_b, noise_r, histT, noise_c)
    rankf = ranks.reshape(_NP)

    xs, sidx = permute_sc(x2, rankf)
    return (xs[None], sidx[None])


# speedup vs baseline: 1.9645x; 1.1107x over previous
import functools

import jax
import jax.numpy as jnp
from jax import lax
from jax.experimental import pallas as pl
from jax.experimental.pallas import tpu as pltpu
from jax.experimental.pallas import tpu_sc as plsc

_N = 10000
_E = 320000
_D = 128
_NP = 10240
_EP = 327680
_PAD_BIN = 10016
_IB = 256
_JC = 1024

def _hist_sc_body(src_hbm, hist_hbm, idx_v, ones_v, zeros_v, hist_sh):
    c = lax.axis_index("c")
    s = lax.axis_index("s")
    w = c * 16 + s
    for k in range(8):
        ones_v[pl.ds(k * 16, 16)] = jnp.full((16,), 1.0, jnp.float32)
    for k in range(40):
        zeros_v[pl.ds(k * 16, 16)] = jnp.zeros((16,), jnp.float32)
    pltpu.sync_copy(zeros_v, hist_sh.at[pl.ds(s * 640, 640)])
    pltpu.sync_copy(src_hbm.at[w], idx_v)
    plsc.subcore_barrier()

    def chunk(j, carry):
        pltpu.sync_copy(ones_v, hist_sh.at[idx_v.at[j]], add=True)
        return carry

    lax.fori_loop(0, 80, chunk, 0)
    plsc.subcore_barrier()
    pltpu.sync_copy(hist_sh.at[pl.ds(s * 640, 640)],
                    hist_hbm.at[c, pl.ds(s * 640, 640)])


def _rank_body(hist_a, hist_b, noise_r, histT, noise_c, row_ref, col_ref,
               colacc):
    g = pl.program_id(0)

    @pl.when(g == 0)
    def _():
        colacc[...] = jnp.zeros((_NP // _JC, _JC), jnp.float32)

    ki = histT[:, 0:1] + histT[:, 1:2] + noise_c[...]
    ii = g * _IB + lax.broadcasted_iota(jnp.int32, (_IB, 1), 0)

    def kj(jc):
        return (hist_a[pl.ds(jc, 1), :] + hist_b[pl.ds(jc, 1), :]
                + noise_r[pl.ds(jc, 1), :])

    def reduce8(cnt, acc):
        for k in range(_JC // 128):
            acc = acc + cnt[:, k * 128:(k + 1) * 128]
        return acc

    def accumulate(jc, cnt, acc):
        colacc[pl.ds(jc, 1), :] = (colacc[pl.ds(jc, 1), :]
                                   + jnp.sum(cnt, axis=0, keepdims=True))
        return reduce8(cnt, acc)

    def le_body(jc, acc):
        return accumulate(jc, jnp.where(kj(jc) <= ki, 1.0, 0.0), acc)

    def mid_body(jc, acc):
        jj = jc * _JC + lax.broadcasted_iota(jnp.int32, (1, _JC), 1)
        cnt = jnp.where((kj(jc) <= ki) & (jj < ii), 1.0, 0.0)
        return accumulate(jc, cnt, acc)

    gd = g // (_JC // _IB)
    acc = jnp.zeros((_IB, 128), jnp.float32)
    acc = lax.fori_loop(0, gd, le_body, acc)
    acc = mid_body(gd, acc)
    row_ref[...] = jnp.sum(acc, axis=1, keepdims=True).astype(jnp.int32)

    @pl.when(g == _NP // _IB - 1)
    def _():
        col_ref[...] = colacc[...].astype(jnp.int32)


_rank_tc = pl.pallas_call(
    _rank_body,
    grid=(_NP // _IB,),
    in_specs=[
        pl.BlockSpec((_NP // _JC, _JC), lambda g: (0, 0)),
        pl.BlockSpec((_NP // _JC, _JC), lambda g: (0, 0)),
        pl.BlockSpec((_NP // _JC, _JC), lambda g: (0, 0)),
        pl.BlockSpec((_IB, 2), lambda g: (g, 0)),
        pl.BlockSpec((_IB, 1), lambda g: (g, 0)),
    ],
    out_specs=[
        pl.BlockSpec((_IB, 1), lambda g: (g, 0)),
        pl.BlockSpec((_NP // _JC, _JC), lambda g: (0, 0)),
    ],
    out_shape=[
        jax.ShapeDtypeStruct((_NP, 1), jnp.int32),
        jax.ShapeDtypeStruct((_NP // _JC, _JC), jnp.int32),
    ],
    scratch_shapes=[pltpu.VMEM((_NP // _JC, _JC), jnp.float32)],
)


def _permute_sc_body(x_hbm, rowf_hbm, colf_hbm, xs_hbm, idx_hbm,
                     rk_v, row_v, col_v, rows_v, vals_v,
                     trk_v, trow_v, tcol_v, trows_v, tvals_v):
    c = lax.axis_index("c")
    s = lax.axis_index("s")
    w = c * 16 + s

    def do_chunk(ci):
        pltpu.sync_copy(rowf_hbm.at[pl.ds(ci * 128, 128)], row_v)
        pltpu.sync_copy(colf_hbm.at[pl.ds(ci * 128, 128)], col_v)
        pltpu.sync_copy(x_hbm.at[pl.ds(ci * 128, 128), :], rows_v)
        for k in range(8):
            sl = pl.ds(k * 16, 16)
            idxs = (ci * 128 + k * 16
                    + lax.broadcasted_iota(jnp.int32, (16,), 0))
            vals_v[sl] = idxs
            rk_v[sl] = row_v[sl] + ((_NP - 1) - idxs) - col_v[sl]
        pltpu.sync_copy(rows_v, xs_hbm.at[rk_v])
        pltpu.sync_copy(vals_v, idx_hbm.at[rk_v])

    do_chunk(w)
    do_chunk(w + 32)

    @pl.when(w < 14)
    def _():
        do_chunk(w + 64)

    @pl.when(w == 14)
    def _():
        pltpu.sync_copy(rowf_hbm.at[pl.ds(9984, 16)], trow_v)
        pltpu.sync_copy(colf_hbm.at[pl.ds(9984, 16)], tcol_v)
        pltpu.sync_copy(x_hbm.at[pl.ds(9984, 16), :], trows_v)
        idxs = 9984 + lax.broadcasted_iota(jnp.int32, (16,), 0)
        tvals_v[...] = idxs
        trk_v[...] = trow_v[...] + ((_NP - 1) - idxs) - tcol_v[...]
        pltpu.sync_copy(trows_v, xs_hbm.at[trk_v])
        pltpu.sync_copy(tvals_v, idx_hbm.at[trk_v])


@functools.lru_cache(maxsize=1)
def _sc_kernels():
    mesh = plsc.VectorSubcoreMesh(core_axis_name="c", subcore_axis_name="s",
                                  num_cores=2, num_subcores=16)
    hist_sc = pl.kernel(
        _hist_sc_body,
        out_type=jax.ShapeDtypeStruct((2, _NP), jnp.float32),
        mesh=mesh,
        scratch_types=[
            pltpu.VMEM((80, 128), jnp.int32),
            pltpu.VMEM((128,), jnp.float32),
            pltpu.VMEM((640,), jnp.float32),
            pltpu.VMEM_SHARED((_NP,), jnp.float32),
        ],
    )
    permute_sc = pl.kernel(
        _permute_sc_body,
        out_type=(jax.ShapeDtypeStruct((_N, _D), jnp.float32),
                  jax.ShapeDtypeStruct((_N,), jnp.int32)),
        mesh=mesh,
        scratch_types=[
            pltpu.VMEM((128,), jnp.int32),
            pltpu.VMEM((128,), jnp.int32),
            pltpu.VMEM((128,), jnp.int32),
            pltpu.VMEM((128, _D), jnp.float32),
            pltpu.VMEM((128,), jnp.int32),
            pltpu.VMEM((16,), jnp.int32),
            pltpu.VMEM((16,), jnp.int32),
            pltpu.VMEM((16,), jnp.int32),
            pltpu.VMEM((16, _D), jnp.float32),
            pltpu.VMEM((16,), jnp.int32),
        ],
    )
    return hist_sc, permute_sc


def kernel(x, edge_index):
    hist_sc, permute_sc = _sc_kernels()
    x2 = x if x.ndim == 2 else x[0]
    src = edge_index[0].astype(jnp.int32)
    src_pad = jnp.concatenate(
        [src, jnp.full((_EP - _E,), _PAD_BIN, jnp.int32)]).reshape(32, 80, 128)

    noise = (jax.random.uniform(jax.random.key(1), (1, _N), dtype=jnp.float32)
             * 0.1)
    noise_pad = jnp.concatenate(
        [noise[0], jnp.full((_NP - _N,), jnp.inf, jnp.float32)])

    hist2 = hist_sc(src_pad)
    hist_a = hist2[0].reshape(_NP // _JC, _JC)
    hist_b = hist2[1].reshape(_NP // _JC, _JC)
    noise_r = noise_pad.reshape(_NP // _JC, _JC)
    histT = hist2.T
    noise_c = noise_pad.reshape(_NP, 1)

    rowc, colc = _rank_tc(hist_a, hist_b, noise_r, histT, noise_c)
    rowf = rowc.reshape(_NP)
    colf = colc.reshape(_NP)

    xs, sidx = permute_sc(x2, rowf, colf)
    return (xs[None], sidx[None])


# speedup vs baseline: 2.1436x; 1.0911x over previous
import functools

import jax
import jax.numpy as jnp
import numpy as np
from jax import lax
from jax.experimental import pallas as pl
from jax.experimental.pallas import tpu as pltpu
from jax.experimental.pallas import tpu_sc as plsc

_N = 10000
_E = 320000
_D = 128
_NP = 10240
_ER = 2500
_IB = 256
_JC = 1024

def _hist_sc_body(edges_hbm, hist_hbm, idx_v, extra_v, ones_v, zeros_v,
                  hist_sh):
    c = lax.axis_index("c")
    s = lax.axis_index("s")
    w = c * 16 + s
    for k in range(8):
        ones_v[pl.ds(k * 16, 16)] = jnp.full((16,), 1.0, jnp.float32)
    for k in range(40):
        zeros_v[pl.ds(k * 16, 16)] = jnp.zeros((16,), jnp.float32)
    pltpu.sync_copy(zeros_v, hist_sh.at[pl.ds(s * 640, 640)])

    @pl.when(w < 31)
    def _():
        pltpu.sync_copy(edges_hbm.at[0, pl.ds(w * 80, 80), :], idx_v)

    @pl.when(w == 31)
    def _():
        pltpu.sync_copy(edges_hbm.at[0, pl.ds(2480, 20), :],
                        idx_v.at[pl.ds(0, 20)])

    plsc.subcore_barrier()

    def chunk(j, carry):
        pltpu.sync_copy(ones_v, hist_sh.at[idx_v.at[j]], add=True)
        return carry

    @pl.when(w < 31)
    def _():
        lax.fori_loop(0, 80, chunk, 0)

    @pl.when(w == 31)
    def _():
        lax.fori_loop(0, 20, chunk, 0)

    plsc.subcore_barrier()
    pltpu.sync_copy(hist_sh.at[pl.ds(s * 640, 640)],
                    hist_hbm.at[c, pl.ds(s * 640, 640)])


def _rank_body(hist_a, hist_b, noise_r, histT, noise_c, row_ref, col_ref,
               colacc):
    g = pl.program_id(0)

    @pl.when(g == 0)
    def _():
        colacc[...] = jnp.zeros((_NP // _JC, _JC), jnp.float32)

    ki = histT[:, 0:1] + histT[:, 1:2] + noise_c[...]
    ii = g * _IB + lax.broadcasted_iota(jnp.int32, (_IB, 1), 0)

    def kj(jc):
        return (hist_a[pl.ds(jc, 1), :] + hist_b[pl.ds(jc, 1), :]
                + noise_r[pl.ds(jc, 1), :])

    def reduce8(cnt, acc):
        for k in range(_JC // 128):
            acc = acc + cnt[:, k * 128:(k + 1) * 128]
        return acc

    def accumulate(jc, cnt, acc):
        colacc[pl.ds(jc, 1), :] = (colacc[pl.ds(jc, 1), :]
                                   + jnp.sum(cnt, axis=0, keepdims=True))
        return reduce8(cnt, acc)

    def le_body(jc, acc):
        return accumulate(jc, jnp.where(kj(jc) <= ki, 1.0, 0.0), acc)

    def mid_body(jc, acc):
        jj = jc * _JC + lax.broadcasted_iota(jnp.int32, (1, _JC), 1)
        cnt = jnp.where((kj(jc) <= ki) & (jj < ii), 1.0, 0.0)
        return accumulate(jc, cnt, acc)

    gd = g // (_JC // _IB)
    acc = jnp.zeros((_IB, 128), jnp.float32)
    acc = lax.fori_loop(0, gd, le_body, acc)
    acc = mid_body(gd, acc)
    row_ref[...] = jnp.sum(acc, axis=1, keepdims=True).astype(jnp.int32)

    @pl.when(g == _NP // _IB - 1)
    def _():
        col_ref[...] = colacc[...].astype(jnp.int32)


_rank_tc = pl.pallas_call(
    _rank_body,
    grid=(_NP // _IB,),
    in_specs=[
        pl.BlockSpec((_NP // _JC, _JC), lambda g: (0, 0)),
        pl.BlockSpec((_NP // _JC, _JC), lambda g: (0, 0)),
        pl.BlockSpec((_NP // _JC, _JC), lambda g: (0, 0)),
        pl.BlockSpec((_IB, 2), lambda g: (g, 0)),
        pl.BlockSpec((_IB, 1), lambda g: (g, 0)),
    ],
    out_specs=[
        pl.BlockSpec((_IB, 1), lambda g: (g, 0)),
        pl.BlockSpec((_NP // _JC, _JC), lambda g: (0, 0)),
    ],
    out_shape=[
        jax.ShapeDtypeStruct((_NP, 1), jnp.int32),
        jax.ShapeDtypeStruct((_NP // _JC, _JC), jnp.int32),
    ],
    scratch_shapes=[pltpu.VMEM((_NP // _JC, _JC), jnp.float32)],
)


def _permute_sc_body(x_hbm, rowf_hbm, colf_hbm, xs_hbm, idx_hbm,
                     rk0, row0, col0, rows0, vals0,
                     rk1, row1, col1, rows1, vals1,
                     trk_v, trow_v, tcol_v, trows_v, tvals_v,
                     seml0, seml1, sems0, sems1):
    c = lax.axis_index("c")
    s = lax.axis_index("s")
    w = c * 16 + s

    def start_loads(ci, row_v, col_v, rows_v, sem):
        return (
            pltpu.async_copy(rowf_hbm.at[pl.ds(ci * 128, 128)], row_v, sem),
            pltpu.async_copy(colf_hbm.at[pl.ds(ci * 128, 128)], col_v, sem),
            pltpu.async_copy(x_hbm.at[pl.ds(ci * 128, 128), :], rows_v, sem),
        )

    def fill_vals(ci, vals_v):
        for k in range(8):
            vals_v[pl.ds(k * 16, 16)] = (
                ci * 128 + k * 16 + lax.broadcasted_iota(jnp.int32, (16,), 0))

    def finish_chunk(loads, row_v, col_v, rows_v, vals_v, rk_v, sem):
        for h in loads:
            h.wait()
        for k in range(8):
            sl = pl.ds(k * 16, 16)
            rk_v[sl] = row_v[sl] + ((_NP - 1) - vals_v[sl]) - col_v[sl]
        return (
            pltpu.async_copy(rows_v, xs_hbm.at[rk_v], sem),
            pltpu.async_copy(vals_v, idx_hbm.at[rk_v], sem),
        )

    l0 = start_loads(w, row0, col0, rows0, seml0)
    l1 = start_loads(w + 32, row1, col1, rows1, seml1)
    fill_vals(w, vals0)
    fill_vals(w + 32, vals1)
    s0 = finish_chunk(l0, row0, col0, rows0, vals0, rk0, sems0)
    s1 = finish_chunk(l1, row1, col1, rows1, vals1, rk1, sems1)

    @pl.when(w < 14)
    def _():
        for h in s0:
            h.wait()
        l2 = start_loads(w + 64, row0, col0, rows0, seml0)
        fill_vals(w + 64, vals0)
        s2 = finish_chunk(l2, row0, col0, rows0, vals0, rk0, sems0)
        for h in s2:
            h.wait()

    @pl.when(w >= 14)
    def _():
        for h in s0:
            h.wait()

    for h in s1:
        h.wait()

    @pl.when(w == 14)
    def _():
        pltpu.sync_copy(rowf_hbm.at[pl.ds(9984, 16)], trow_v)
        pltpu.sync_copy(colf_hbm.at[pl.ds(9984, 16)], tcol_v)
        pltpu.sync_copy(x_hbm.at[pl.ds(9984, 16), :], trows_v)
        idxs = 9984 + lax.broadcasted_iota(jnp.int32, (16,), 0)
        tvals_v[...] = idxs
        trk_v[...] = trow_v[...] + ((_NP - 1) - idxs) - tcol_v[...]
        pltpu.sync_copy(trows_v, xs_hbm.at[trk_v])
        pltpu.sync_copy(tvals_v, idx_hbm.at[trk_v])


@functools.lru_cache(maxsize=1)
def _sc_kernels():
    mesh = plsc.VectorSubcoreMesh(core_axis_name="c", subcore_axis_name="s",
                                  num_cores=2, num_subcores=16)
    hist_sc = pl.kernel(
        _hist_sc_body,
        out_type=jax.ShapeDtypeStruct((2, _NP), jnp.float32),
        mesh=mesh,
        scratch_types=[
            pltpu.VMEM((80, 128), jnp.int32),
            pltpu.VMEM((1, 128), jnp.int32),
            pltpu.VMEM((128,), jnp.float32),
            pltpu.VMEM((640,), jnp.float32),
            pltpu.VMEM_SHARED((_NP,), jnp.float32),
        ],
    )
    chunk_bufs = [
        pltpu.VMEM((128,), jnp.int32),
        pltpu.VMEM((128,), jnp.int32),
        pltpu.VMEM((128,), jnp.int32),
        pltpu.VMEM((128, _D), jnp.float32),
        pltpu.VMEM((128,), jnp.int32),
    ]
    permute_sc = pl.kernel(
        _permute_sc_body,
        out_type=(jax.ShapeDtypeStruct((_N, _D), jnp.float32),
                  jax.ShapeDtypeStruct((_N,), jnp.int32)),
        mesh=mesh,
        scratch_types=chunk_bufs + chunk_bufs + [
            pltpu.VMEM((16,), jnp.int32),
            pltpu.VMEM((16,), jnp.int32),
            pltpu.VMEM((16,), jnp.int32),
            pltpu.VMEM((16, _D), jnp.float32),
            pltpu.VMEM((16,), jnp.int32),
            pltpu.SemaphoreType.DMA,
            pltpu.SemaphoreType.DMA,
            pltpu.SemaphoreType.DMA,
            pltpu.SemaphoreType.DMA,
        ],
    )
    return hist_sc, permute_sc


_noise_cache = []


def _noise_pad_const():
    if not _noise_cache:
        with jax.ensure_compile_time_eval():
            noise = (jax.random.uniform(jax.random.key(1), (1, _N),
                                        dtype=jnp.float32) * 0.1)
            arr = np.asarray(noise[0])
        _noise_cache.append(np.concatenate(
            [arr, np.full(_NP - _N, np.inf, np.float32)]))
    return _noise_cache[0]


def kernel(x, edge_index):
    hist_sc, permute_sc = _sc_kernels()
    x2 = x if x.ndim == 2 else x[0]
    edges = edge_index.astype(jnp.int32).reshape(2, _ER, 128)
    noise_pad = jnp.asarray(_noise_pad_const())

    hist2 = hist_sc(edges)
    hist_a = hist2[0].reshape(_NP // _JC, _JC)
    hist_b = hist2[1].reshape(_NP // _JC, _JC)
    noise_r = noise_pad.reshape(_NP // _JC, _JC)
    histT = hist2.T
    noise_c = noise_pad.reshape(_NP, 1)

    rowc, colc = _rank_tc(hist_a, hist_b, noise_r, histT, noise_c)
    rowf = rowc.reshape(_NP)
    colf = colc.reshape(_NP)

    xs, sidx = permute_sc(x2, rowf, colf)
    return (xs[None], sidx[None])


# speedup vs baseline: 2.1469x; 1.0016x over previous
import functools

import jax
import jax.numpy as jnp
import numpy as np
from jax import lax
from jax.experimental import pallas as pl
from jax.experimental.pallas import tpu as pltpu
from jax.experimental.pallas import tpu_sc as plsc

_N = 10000
_E = 320000
_D = 128
_NP = 10240
_ER = 2500
_IB = 256
_JC = 1024

def _hist_sc_body(edges_hbm, hist_hbm, idx_v, extra_v, ones_v, zeros_v,
                  hist_sh):
    c = lax.axis_index("c")
    s = lax.axis_index("s")
    w = c * 16 + s
    for k in range(8):
        ones_v[pl.ds(k * 16, 16)] = jnp.full((16,), 1.0, jnp.float32)
    for k in range(40):
        zeros_v[pl.ds(k * 16, 16)] = jnp.zeros((16,), jnp.float32)
    pltpu.sync_copy(zeros_v, hist_sh.at[pl.ds(s * 640, 640)])

    @pl.when(w < 31)
    def _():
        pltpu.sync_copy(edges_hbm.at[0, pl.ds(w * 80, 80), :], idx_v)

    @pl.when(w == 31)
    def _():
        pltpu.sync_copy(edges_hbm.at[0, pl.ds(2480, 20), :],
                        idx_v.at[pl.ds(0, 20)])

    plsc.subcore_barrier()

    def chunk(j, carry):
        pltpu.sync_copy(ones_v, hist_sh.at[idx_v.at[j]], add=True)
        return carry

    @pl.when(w < 31)
    def _():
        lax.fori_loop(0, 80, chunk, 0)

    @pl.when(w == 31)
    def _():
        lax.fori_loop(0, 20, chunk, 0)

    plsc.subcore_barrier()
    pltpu.sync_copy(hist_sh.at[pl.ds(s * 640, 640)],
                    hist_hbm.at[c, pl.ds(s * 640, 640)])


def _rank_body(hist_a, hist_b, noise_r, histT, noise_c, row_ref, col_ref,
               colacc):
    g = pl.program_id(0)

    @pl.when(g == 0)
    def _():
        colacc[...] = jnp.zeros((_NP // _JC, _JC), jnp.float32)

    ki = histT[:, 0:1] + histT[:, 1:2] + noise_c[...]
    ii = g * _IB + lax.broadcasted_iota(jnp.int32, (_IB, 1), 0)

    def kj(jc):
        return (hist_a[pl.ds(jc, 1), :] + hist_b[pl.ds(jc, 1), :]
                + noise_r[pl.ds(jc, 1), :])

    def reduce8(cnt, acc):
        for k in range(_JC // 128):
            acc = acc + cnt[:, k * 128:(k + 1) * 128]
        return acc

    def accumulate(jc, cnt, acc):
        colacc[pl.ds(jc, 1), :] = (colacc[pl.ds(jc, 1), :]
                                   + jnp.sum(cnt, axis=0, keepdims=True))
        return reduce8(cnt, acc)

    def le_body(jc, acc):
        return accumulate(jc, jnp.where(kj(jc) <= ki, 1.0, 0.0), acc)

    def mid_body(jc, acc):
        jj = jc * _JC + lax.broadcasted_iota(jnp.int32, (1, _JC), 1)
        cnt = jnp.where((kj(jc) <= ki) & (jj < ii), 1.0, 0.0)
        return accumulate(jc, cnt, acc)

    gd = g // (_JC // _IB)
    acc = jnp.zeros((_IB, 128), jnp.float32)
    acc = lax.fori_loop(0, gd, le_body, acc)
    acc = mid_body(gd, acc)
    row_ref[...] = jnp.sum(acc, axis=1, keepdims=True).astype(jnp.int32)

    @pl.when(g == _NP // _IB - 1)
    def _():
        col_ref[...] = colacc[...].astype(jnp.int32)


_rank_tc = pl.pallas_call(
    _rank_body,
    grid=(_NP // _IB,),
    in_specs=[
        pl.BlockSpec((_NP // _JC, _JC), lambda g: (0, 0)),
        pl.BlockSpec((_NP // _JC, _JC), lambda g: (0, 0)),
        pl.BlockSpec((_NP // _JC, _JC), lambda g: (0, 0)),
        pl.BlockSpec((_IB, 2), lambda g: (g, 0)),
        pl.BlockSpec((_IB, 1), lambda g: (g, 0)),
    ],
    out_specs=[
        pl.BlockSpec((_IB, 1), lambda g: (g, 0)),
        pl.BlockSpec((_NP // _JC, _JC), lambda g: (0, 0)),
    ],
    out_shape=[
        jax.ShapeDtypeStruct((_NP, 1), jnp.int32),
        jax.ShapeDtypeStruct((_NP // _JC, _JC), jnp.int32),
    ],
    scratch_shapes=[pltpu.VMEM((_NP // _JC, _JC), jnp.float32)],
)


def _permute_sc_body(x_hbm, rowf_hbm, colf_hbm, xs_hbm, idx_hbm,
                     rk0, row0, col0, rows0, vals0,
                     rk1, row1, col1, rows1, vals1,
                     rk2, row2, col2, rows2, vals2,
                     trk_v, trow_v, tcol_v, trows_v, tvals_v,
                     seml0, seml1, seml2, sems0, sems1, sems2):
    c = lax.axis_index("c")
    s = lax.axis_index("s")
    w = c * 16 + s
    sets = [
        (rk0, row0, col0, rows0, vals0, seml0, sems0),
        (rk1, row1, col1, rows1, vals1, seml1, sems1),
        (rk2, row2, col2, rows2, vals2, seml2, sems2),
    ]

    def load_descrs(ci, t):
        rk_v, row_v, col_v, rows_v, vals_v, seml, sems = sets[t]
        return (
            pltpu.make_async_copy(rowf_hbm.at[pl.ds(ci * 128, 128)], row_v,
                                  seml),
            pltpu.make_async_copy(colf_hbm.at[pl.ds(ci * 128, 128)], col_v,
                                  seml),
            pltpu.make_async_copy(x_hbm.at[pl.ds(ci * 128, 128), :], rows_v,
                                  seml),
        )

    def scat_descrs(t):
        rk_v, row_v, col_v, rows_v, vals_v, seml, sems = sets[t]
        return (
            pltpu.make_async_copy(rows_v, xs_hbm.at[rk_v], sems),
            pltpu.make_async_copy(vals_v, idx_hbm.at[rk_v], sems),
        )

    def finish_chunk(ci, t):
        rk_v, row_v, col_v, rows_v, vals_v, seml, sems = sets[t]
        for d in load_descrs(ci, t):
            d.wait()
        for k in range(8):
            sl = pl.ds(k * 16, 16)
            idxs = (ci * 128 + k * 16
                    + lax.broadcasted_iota(jnp.int32, (16,), 0))
            vals_v[sl] = idxs
            rk_v[sl] = row_v[sl] + ((_NP - 1) - idxs) - col_v[sl]
        for d in scat_descrs(t):
            d.start()

    for d in load_descrs(w, 0):
        d.start()
    for d in load_descrs(w + 32, 1):
        d.start()

    @pl.when(w < 14)
    def _():
        for d in load_descrs(w + 64, 2):
            d.start()

    finish_chunk(w, 0)
    finish_chunk(w + 32, 1)

    @pl.when(w < 14)
    def _():
        finish_chunk(w + 64, 2)
        for d in scat_descrs(2):
            d.wait()

    for d in scat_descrs(0):
        d.wait()
    for d in scat_descrs(1):
        d.wait()

    @pl.when(w == 14)
    def _():
        pltpu.sync_copy(rowf_hbm.at[pl.ds(9984, 16)], trow_v)
        pltpu.sync_copy(colf_hbm.at[pl.ds(9984, 16)], tcol_v)
        pltpu.sync_copy(x_hbm.at[pl.ds(9984, 16), :], trows_v)
        idxs = 9984 + lax.broadcasted_iota(jnp.int32, (16,), 0)
        tvals_v[...] = idxs
        trk_v[...] = trow_v[...] + ((_NP - 1) - idxs) - tcol_v[...]
        pltpu.sync_copy(trows_v, xs_hbm.at[trk_v])
        pltpu.sync_copy(tvals_v, idx_hbm.at[trk_v])


@functools.lru_cache(maxsize=1)
def _sc_kernels():
    mesh = plsc.VectorSubcoreMesh(core_axis_name="c", subcore_axis_name="s",
                                  num_cores=2, num_subcores=16)
    hist_sc = pl.kernel(
        _hist_sc_body,
        out_type=jax.ShapeDtypeStruct((2, _NP), jnp.float32),
        mesh=mesh,
        scratch_types=[
            pltpu.VMEM((80, 128), jnp.int32),
            pltpu.VMEM((1, 128), jnp.int32),
            pltpu.VMEM((128,), jnp.float32),
            pltpu.VMEM((640,), jnp.float32),
            pltpu.VMEM_SHARED((_NP,), jnp.float32),
        ],
    )
    chunk_bufs = [
        pltpu.VMEM((128,), jnp.int32),
        pltpu.VMEM((128,), jnp.int32),
        pltpu.VMEM((128,), jnp.int32),
        pltpu.VMEM((128, _D), jnp.float32),
        pltpu.VMEM((128,), jnp.int32),
    ]
    permute_sc = pl.kernel(
        _permute_sc_body,
        out_type=(jax.ShapeDtypeStruct((_N, _D), jnp.float32),
                  jax.ShapeDtypeStruct((_N,), jnp.int32)),
        mesh=mesh,
        scratch_types=chunk_bufs + chunk_bufs + chunk_bufs + [
            pltpu.VMEM((16,), jnp.int32),
            pltpu.VMEM((16,), jnp.int32),
            pltpu.VMEM((16,), jnp.int32),
            pltpu.VMEM((16, _D), jnp.float32),
            pltpu.VMEM((16,), jnp.int32),
            pltpu.SemaphoreType.DMA,
            pltpu.SemaphoreType.DMA,
            pltpu.SemaphoreType.DMA,
            pltpu.SemaphoreType.DMA,
            pltpu.SemaphoreType.DMA,
            pltpu.SemaphoreType.DMA,
        ],
    )
    return hist_sc, permute_sc


_noise_cache = []


def _noise_pad_const():
    if not _noise_cache:
        with jax.ensure_compile_time_eval():
            noise = (jax.random.uniform(jax.random.key(1), (1, _N),
                                        dtype=jnp.float32) * 0.1)
            arr = np.asarray(noise[0])
        _noise_cache.append(np.concatenate(
            [arr, np.full(_NP - _N, np.inf, np.float32)]))
    return _noise_cache[0]


def kernel(x, edge_index):
    hist_sc, permute_sc = _sc_kernels()
    x2 = x if x.ndim == 2 else x[0]
    edges = edge_index.astype(jnp.int32).reshape(2, _ER, 128)
    noise_pad = jnp.asarray(_noise_pad_const())

    hist2 = hist_sc(edges)
    hist_a = hist2[0].reshape(_NP // _JC, _JC)
    hist_b = hist2[1].reshape(_NP // _JC, _JC)
    noise_r = noise_pad.reshape(_NP // _JC, _JC)
    histT = hist2.T
    noise_c = noise_pad.reshape(_NP, 1)

    rowc, colc = _rank_tc(hist_a, hist_b, noise_r, histT, noise_c)
    rowf = rowc.reshape(_NP)
    colf = colc.reshape(_NP)

    xs, sidx = permute_sc(x2, rowf, colf)
    return (xs[None], sidx[None])


# speedup vs baseline: 2.2349x; 1.0410x over previous
import functools

import jax
import jax.numpy as jnp
import numpy as np
from jax import lax
from jax.experimental import pallas as pl
from jax.experimental.pallas import tpu as pltpu
from jax.experimental.pallas import tpu_sc as plsc

_N = 10000
_E = 320000
_D = 128
_NP = 10240
_ER = 2500
_IB = 256
_JC = 1024

def _hist_sc_body(edges_hbm, hist_hbm, idx_v, extra_v, ones_v, zeros_v,
                  hist_sh):
    c = lax.axis_index("c")
    s = lax.axis_index("s")
    w = c * 16 + s
    for k in range(8):
        ones_v[pl.ds(k * 16, 16)] = jnp.full((16,), 1.0, jnp.float32)
    for k in range(40):
        zeros_v[pl.ds(k * 16, 16)] = jnp.zeros((16,), jnp.float32)
    pltpu.sync_copy(zeros_v, hist_sh.at[pl.ds(s * 640, 640)])

    @pl.when(w < 31)
    def _():
        pltpu.sync_copy(edges_hbm.at[0, pl.ds(w * 80, 80), :], idx_v)

    @pl.when(w == 31)
    def _():
        pltpu.sync_copy(edges_hbm.at[0, pl.ds(2480, 20), :],
                        idx_v.at[pl.ds(0, 20)])

    plsc.subcore_barrier()

    def chunk(j, carry):
        pltpu.sync_copy(ones_v, hist_sh.at[idx_v.at[j]], add=True)
        return carry

    @pl.when(w < 31)
    def _():
        lax.fori_loop(0, 80, chunk, 0)

    @pl.when(w == 31)
    def _():
        lax.fori_loop(0, 20, chunk, 0)

    plsc.subcore_barrier()
    pltpu.sync_copy(hist_sh.at[pl.ds(s * 640, 640)],
                    hist_hbm.at[c, pl.ds(s * 640, 640)])


def _rank_body(hist_a, hist_b, noise_r, histT, noise_c, row_ref, col_ref,
               colacc):
    g = pl.program_id(0)

    @pl.when(g == 0)
    def _():
        colacc[...] = jnp.zeros((_NP // _JC, _JC), jnp.float32)

    ki = histT[:, 0:1] + histT[:, 1:2] + noise_c[...]
    ii = g * _IB + lax.broadcasted_iota(jnp.int32, (_IB, 1), 0)

    def kj(jc):
        return (hist_a[pl.ds(jc, 1), :] + hist_b[pl.ds(jc, 1), :]
                + noise_r[pl.ds(jc, 1), :])

    def reduce8(cnt, acc):
        for k in range(_JC // 128):
            acc = acc + cnt[:, k * 128:(k + 1) * 128]
        return acc

    def accumulate(jc, cnt, acc):
        colacc[pl.ds(jc, 1), :] = (colacc[pl.ds(jc, 1), :]
                                   + jnp.sum(cnt, axis=0, keepdims=True))
        return reduce8(cnt, acc)

    def le_body(jc, acc):
        return accumulate(jc, jnp.where(kj(jc) <= ki, 1.0, 0.0), acc)

    def mid_body(jc, acc):
        jj = jc * _JC + lax.broadcasted_iota(jnp.int32, (1, _JC), 1)
        cnt = jnp.where((kj(jc) <= ki) & (jj < ii), 1.0, 0.0)
        return accumulate(jc, cnt, acc)

    def le_body2(p, acc):
        return le_body(2 * p + 1, le_body(2 * p, acc))

    gd = g // (_JC // _IB)
    acc = jnp.zeros((_IB, 128), jnp.float32)
    acc = lax.fori_loop(0, gd // 2, le_body2, acc)
    acc = lax.fori_loop((gd // 2) * 2, gd, le_body, acc)
    acc = mid_body(gd, acc)
    row_ref[...] = jnp.sum(acc, axis=1, keepdims=True).astype(jnp.int32)

    @pl.when(g == _NP // _IB - 1)
    def _():
        col_ref[...] = colacc[...].astype(jnp.int32)


_rank_tc = pl.pallas_call(
    _rank_body,
    grid=(_NP // _IB,),
    in_specs=[
        pl.BlockSpec((_NP // _JC, _JC), lambda g: (0, 0)),
        pl.BlockSpec((_NP // _JC, _JC), lambda g: (0, 0)),
        pl.BlockSpec((_NP // _JC, _JC), lambda g: (0, 0)),
        pl.BlockSpec((_IB, 2), lambda g: (g, 0)),
        pl.BlockSpec((_IB, 1), lambda g: (g, 0)),
    ],
    out_specs=[
        pl.BlockSpec((_IB, 1), lambda g: (g, 0)),
        pl.BlockSpec((_NP // _JC, _JC), lambda g: (0, 0)),
    ],
    out_shape=[
        jax.ShapeDtypeStruct((_NP, 1), jnp.int32),
        jax.ShapeDtypeStruct((_NP // _JC, _JC), jnp.int32),
    ],
    scratch_shapes=[pltpu.VMEM((_NP // _JC, _JC), jnp.float32)],
)


def _permute_sc_body(x_hbm, rowf_hbm, colf_hbm, xs_hbm, idx_hbm,
                     rk0, row0, col0, rows0, vals0,
                     rk1, row1, col1, rows1, vals1,
                     rk2, row2, col2, rows2, vals2,
                     trk_v, trow_v, tcol_v, trows_v, tvals_v,
                     seml0, seml1, seml2, sems0, sems1, sems2):
    c = lax.axis_index("c")
    s = lax.axis_index("s")
    w = c * 16 + s
    sets = [
        (rk0, row0, col0, rows0, vals0, seml0, sems0),
        (rk1, row1, col1, rows1, vals1, seml1, sems1),
        (rk2, row2, col2, rows2, vals2, seml2, sems2),
    ]

    def load_descrs(ci, t):
        rk_v, row_v, col_v, rows_v, vals_v, seml, sems = sets[t]
        return (
            pltpu.make_async_copy(rowf_hbm.at[pl.ds(ci * 128, 128)], row_v,
                                  seml),
            pltpu.make_async_copy(colf_hbm.at[pl.ds(ci * 128, 128)], col_v,
                                  seml),
            pltpu.make_async_copy(x_hbm.at[pl.ds(ci * 128, 128), :], rows_v,
                                  seml),
        )

    def scat_descrs(t):
        rk_v, row_v, col_v, rows_v, vals_v, seml, sems = sets[t]
        return (
            pltpu.make_async_copy(rows_v, xs_hbm.at[rk_v], sems),
            pltpu.make_async_copy(vals_v, idx_hbm.at[rk_v], sems),
        )

    def finish_chunk(ci, t):
        rk_v, row_v, col_v, rows_v, vals_v, seml, sems = sets[t]
        for d in load_descrs(ci, t):
            d.wait()
        for k in range(8):
            sl = pl.ds(k * 16, 16)
            idxs = (ci * 128 + k * 16
                    + lax.broadcasted_iota(jnp.int32, (16,), 0))
            vals_v[sl] = idxs
            rk_v[sl] = row_v[sl] + ((_NP - 1) - idxs) - col_v[sl]
        for d in scat_descrs(t):
            d.start()

    for d in load_descrs(w, 0):
        d.start()
    for d in load_descrs(w + 32, 1):
        d.start()

    @pl.when(w < 14)
    def _():
        for d in load_descrs(w + 64, 2):
            d.start()

    finish_chunk(w, 0)
    finish_chunk(w + 32, 1)

    @pl.when(w < 14)
    def _():
        finish_chunk(w + 64, 2)
        for d in scat_descrs(2):
            d.wait()

    for d in scat_descrs(0):
        d.wait()
    for d in scat_descrs(1):
        d.wait()

    @pl.when(w == 14)
    def _():
        pltpu.sync_copy(rowf_hbm.at[pl.ds(9984, 16)], trow_v)
        pltpu.sync_copy(colf_hbm.at[pl.ds(9984, 16)], tcol_v)
        pltpu.sync_copy(x_hbm.at[pl.ds(9984, 16), :], trows_v)
        idxs = 9984 + lax.broadcasted_iota(jnp.int32, (16,), 0)
        tvals_v[...] = idxs
        trk_v[...] = trow_v[...] + ((_NP - 1) - idxs) - tcol_v[...]
        pltpu.sync_copy(trows_v, xs_hbm.at[trk_v])
        pltpu.sync_copy(tvals_v, idx_hbm.at[trk_v])


@functools.lru_cache(maxsize=1)
def _sc_kernels():
    mesh = plsc.VectorSubcoreMesh(core_axis_name="c", subcore_axis_name="s",
                                  num_cores=2, num_subcores=16)
    hist_sc = pl.kernel(
        _hist_sc_body,
        out_type=jax.ShapeDtypeStruct((2, _NP), jnp.float32),
        mesh=mesh,
        scratch_types=[
            pltpu.VMEM((80, 128), jnp.int32),
            pltpu.VMEM((1, 128), jnp.int32),
            pltpu.VMEM((128,), jnp.float32),
            pltpu.VMEM((640,), jnp.float32),
            pltpu.VMEM_SHARED((_NP,), jnp.float32),
        ],
    )
    chunk_bufs = [
        pltpu.VMEM((128,), jnp.int32),
        pltpu.VMEM((128,), jnp.int32),
        pltpu.VMEM((128,), jnp.int32),
        pltpu.VMEM((128, _D), jnp.float32),
        pltpu.VMEM((128,), jnp.int32),
    ]
    permute_sc = pl.kernel(
        _permute_sc_body,
        out_type=(jax.ShapeDtypeStruct((_N, _D), jnp.float32),
                  jax.ShapeDtypeStruct((_N,), jnp.int32)),
        mesh=mesh,
        scratch_types=chunk_bufs + chunk_bufs + chunk_bufs + [
            pltpu.VMEM((16,), jnp.int32),
            pltpu.VMEM((16,), jnp.int32),
            pltpu.VMEM((16,), jnp.int32),
            pltpu.VMEM((16, _D), jnp.float32),
            pltpu.VMEM((16,), jnp.int32),
            pltpu.SemaphoreType.DMA,
            pltpu.SemaphoreType.DMA,
            pltpu.SemaphoreType.DMA,
            pltpu.SemaphoreType.DMA,
            pltpu.SemaphoreType.DMA,
            pltpu.SemaphoreType.DMA,
        ],
    )
    return hist_sc, permute_sc


_noise_cache = []


def _noise_pad_const():
    if not _noise_cache:
        with jax.ensure_compile_time_eval():
            noise = (jax.random.uniform(jax.random.key(1), (1, _N),
                                        dtype=jnp.float32) * 0.1)
            arr = np.asarray(noise[0])
        _noise_cache.append(np.concatenate(
            [arr, np.full(_NP - _N, np.inf, np.float32)]))
    return _noise_cache[0]


def kernel(x, edge_index):
    hist_sc, permute_sc = _sc_kernels()
    x2 = x if x.ndim == 2 else x[0]
    edges = edge_index.astype(jnp.int32).reshape(2, _ER, 128)
    noise_pad = jnp.asarray(_noise_pad_const())

    hist2 = hist_sc(edges)
    hist_a = hist2[0].reshape(_NP // _JC, _JC)
    hist_b = hist2[1].reshape(_NP // _JC, _JC)
    noise_r = noise_pad.reshape(_NP // _JC, _JC)
    histT = hist2.T
    noise_c = noise_pad.reshape(_NP, 1)

    rowc, colc = _rank_tc(hist_a, hist_b, noise_r, histT, noise_c)
    rowf = rowc.reshape(_NP)
    colf = colc.reshape(_NP)

    xs, sidx = permute_sc(x2, rowf, colf)
    return (xs[None], sidx[None])


# speedup vs baseline: 2.2478x; 1.0058x over previous
import functools

import jax
import jax.numpy as jnp
import numpy as np
from jax import lax
from jax.experimental import pallas as pl
from jax.experimental.pallas import tpu as pltpu
from jax.experimental.pallas import tpu_sc as plsc

_N = 10000
_E = 320000
_D = 128
_NP = 10240
_ER = 2500
_IB = 256
_JC = 1024

def _hist_sc_body(edges_hbm, hist_hbm, idx_v, extra_v, ones_v, zeros_v,
                  hist_sh):
    c = lax.axis_index("c")
    s = lax.axis_index("s")
    w = c * 16 + s
    for k in range(8):
        ones_v[pl.ds(k * 16, 16)] = jnp.full((16,), 1.0, jnp.float32)
    for k in range(40):
        zeros_v[pl.ds(k * 16, 16)] = jnp.zeros((16,), jnp.float32)
    pltpu.sync_copy(zeros_v, hist_sh.at[pl.ds(s * 640, 640)])

    @pl.when(w < 31)
    def _():
        pltpu.sync_copy(edges_hbm.at[0, pl.ds(w * 80, 80), :], idx_v)

    @pl.when(w == 31)
    def _():
        pltpu.sync_copy(edges_hbm.at[0, pl.ds(2480, 20), :],
                        idx_v.at[pl.ds(0, 20)])

    plsc.subcore_barrier()

    def chunk(j, carry):
        pltpu.sync_copy(ones_v, hist_sh.at[idx_v.at[j]], add=True)
        return carry

    @pl.when(w < 31)
    def _():
        lax.fori_loop(0, 80, chunk, 0)

    @pl.when(w == 31)
    def _():
        lax.fori_loop(0, 20, chunk, 0)

    plsc.subcore_barrier()

    @pl.when(s < 10)
    def _():
        pltpu.sync_copy(hist_sh.at[pl.ds(s * 1024, 1024)], hist_hbm.at[c, s])


def _rank_body(hist3, noise_r, histT, noise_c, row_ref, col_ref,
               colacc):
    g = pl.program_id(0)

    @pl.when(g == 0)
    def _():
        colacc[...] = jnp.zeros((_NP // _JC, _JC), jnp.float32)

    ki = histT[:, 0:1] + histT[:, 1:2] + noise_c[...]
    ii = g * _IB + lax.broadcasted_iota(jnp.int32, (_IB, 1), 0)

    def kj(jc):
        return (hist3[0, pl.ds(jc, 1), :] + hist3[1, pl.ds(jc, 1), :]
                + noise_r[pl.ds(jc, 1), :])

    def reduce8(cnt, acc):
        for k in range(_JC // 128):
            acc = acc + cnt[:, k * 128:(k + 1) * 128]
        return acc

    def accumulate(jc, cnt, acc):
        colacc[pl.ds(jc, 1), :] = (colacc[pl.ds(jc, 1), :]
                                   + jnp.sum(cnt, axis=0, keepdims=True))
        return reduce8(cnt, acc)

    def le_body(jc, acc):
        return accumulate(jc, jnp.where(kj(jc) <= ki, 1.0, 0.0), acc)

    def mid_body(jc, acc):
        jj = jc * _JC + lax.broadcasted_iota(jnp.int32, (1, _JC), 1)
        cnt = jnp.where((kj(jc) <= ki) & (jj < ii), 1.0, 0.0)
        return accumulate(jc, cnt, acc)

    def le_body4(p, acc):
        for q in range(4):
            acc = le_body(4 * p + q, acc)
        return acc

    gd = g // (_JC // _IB)
    acc = jnp.zeros((_IB, 128), jnp.float32)
    acc = lax.fori_loop(0, gd // 4, le_body4, acc)
    acc = lax.fori_loop((gd // 4) * 4, gd, le_body, acc)
    acc = mid_body(gd, acc)
    row_ref[...] = jnp.sum(acc, axis=1, keepdims=True).astype(jnp.int32)

    @pl.when(g == _NP // _IB - 1)
    def _():
        col_ref[...] = colacc[...].astype(jnp.int32)


_rank_tc = pl.pallas_call(
    _rank_body,
    grid=(_NP // _IB,),
    in_specs=[
        pl.BlockSpec((2, _NP // _JC, _JC), lambda g: (0, 0, 0)),
        pl.BlockSpec((_NP // _JC, _JC), lambda g: (0, 0)),
        pl.BlockSpec((_IB, 2), lambda g: (g, 0)),
        pl.BlockSpec((_IB, 1), lambda g: (g, 0)),
    ],
    out_specs=[
        pl.BlockSpec((_IB, 1), lambda g: (g, 0)),
        pl.BlockSpec((_NP // _JC, _JC), lambda g: (0, 0)),
    ],
    out_shape=[
        jax.ShapeDtypeStruct((_NP, 1), jnp.int32),
        jax.ShapeDtypeStruct((_NP // _JC, _JC), jnp.int32),
    ],
    scratch_shapes=[pltpu.VMEM((_NP // _JC, _JC), jnp.float32)],
)


def _permute_sc_body(x_hbm, rowf_hbm, colf_hbm, xs_hbm, idx_hbm,
                     rk0, row0, col0, rows0, vals0,
                     rk1, row1, col1, rows1, vals1,
                     rk2, row2, col2, rows2, vals2,
                     trk_v, trow_v, tcol_v, trows_v, tvals_v,
                     seml0, seml1, seml2, sems0, sems1, sems2):
    c = lax.axis_index("c")
    s = lax.axis_index("s")
    w = c * 16 + s
    sets = [
        (rk0, row0, col0, rows0, vals0, seml0, sems0),
        (rk1, row1, col1, rows1, vals1, seml1, sems1),
        (rk2, row2, col2, rows2, vals2, seml2, sems2),
    ]

    def load_descrs(ci, t):
        rk_v, row_v, col_v, rows_v, vals_v, seml, sems = sets[t]
        return (
            pltpu.make_async_copy(rowf_hbm.at[pl.ds(ci * 128, 128)], row_v,
                                  seml),
            pltpu.make_async_copy(colf_hbm.at[pl.ds(ci * 128, 128)], col_v,
                                  seml),
            pltpu.make_async_copy(x_hbm.at[pl.ds(ci * 128, 128), :], rows_v,
                                  seml),
        )

    def scat_descrs(t):
        rk_v, row_v, col_v, rows_v, vals_v, seml, sems = sets[t]
        return (
            pltpu.make_async_copy(rows_v, xs_hbm.at[rk_v], sems),
            pltpu.make_async_copy(vals_v, idx_hbm.at[rk_v], sems),
        )

    def finish_chunk(ci, t):
        rk_v, row_v, col_v, rows_v, vals_v, seml, sems = sets[t]
        for d in load_descrs(ci, t):
            d.wait()
        for k in range(8):
            sl = pl.ds(k * 16, 16)
            idxs = (ci * 128 + k * 16
                    + lax.broadcasted_iota(jnp.int32, (16,), 0))
            vals_v[sl] = idxs
            rk_v[sl] = row_v[sl] + ((_NP - 1) - idxs) - col_v[sl]
        for d in scat_descrs(t):
            d.start()

    for d in load_descrs(w, 0):
        d.start()
    for d in load_descrs(w + 32, 1):
        d.start()

    @pl.when(w < 14)
    def _():
        for d in load_descrs(w + 64, 2):
            d.start()

    finish_chunk(w, 0)
    finish_chunk(w + 32, 1)

    @pl.when(w < 14)
    def _():
        finish_chunk(w + 64, 2)
        for d in scat_descrs(2):
            d.wait()

    for d in scat_descrs(0):
        d.wait()
    for d in scat_descrs(1):
        d.wait()

    @pl.when(w == 14)
    def _():
        pltpu.sync_copy(rowf_hbm.at[pl.ds(9984, 16)], trow_v)
        pltpu.sync_copy(colf_hbm.at[pl.ds(9984, 16)], tcol_v)
        pltpu.sync_copy(x_hbm.at[pl.ds(9984, 16), :], trows_v)
        idxs = 9984 + lax.broadcasted_iota(jnp.int32, (16,), 0)
        tvals_v[...] = idxs
        trk_v[...] = trow_v[...] + ((_NP - 1) - idxs) - tcol_v[...]
        pltpu.sync_copy(trows_v, xs_hbm.at[trk_v])
        pltpu.sync_copy(tvals_v, idx_hbm.at[trk_v])


@functools.lru_cache(maxsize=1)
def _sc_kernels():
    mesh = plsc.VectorSubcoreMesh(core_axis_name="c", subcore_axis_name="s",
                                  num_cores=2, num_subcores=16)
    hist_sc = pl.kernel(
        _hist_sc_body,
        out_type=jax.ShapeDtypeStruct((2, _NP // _JC, _JC), jnp.float32),
        mesh=mesh,
        scratch_types=[
            pltpu.VMEM((80, 128), jnp.int32),
            pltpu.VMEM((1, 128), jnp.int32),
            pltpu.VMEM((128,), jnp.float32),
            pltpu.VMEM((640,), jnp.float32),
            pltpu.VMEM_SHARED((_NP,), jnp.float32),
        ],
    )
    chunk_bufs = [
        pltpu.VMEM((128,), jnp.int32),
        pltpu.VMEM((128,), jnp.int32),
        pltpu.VMEM((128,), jnp.int32),
        pltpu.VMEM((128, _D), jnp.float32),
        pltpu.VMEM((128,), jnp.int32),
    ]
    permute_sc = pl.kernel(
        _permute_sc_body,
        out_type=(jax.ShapeDtypeStruct((_N, _D), jnp.float32),
                  jax.ShapeDtypeStruct((_N,), jnp.int32)),
        mesh=mesh,
        scratch_types=chunk_bufs + chunk_bufs + chunk_bufs + [
            pltpu.VMEM((16,), jnp.int32),
            pltpu.VMEM((16,), jnp.int32),
            pltpu.VMEM((16,), jnp.int32),
            pltpu.VMEM((16, _D), jnp.float32),
            pltpu.VMEM((16,), jnp.int32),
            pltpu.SemaphoreType.DMA,
            pltpu.SemaphoreType.DMA,
            pltpu.SemaphoreType.DMA,
            pltpu.SemaphoreType.DMA,
            pltpu.SemaphoreType.DMA,
            pltpu.SemaphoreType.DMA,
        ],
    )
    return hist_sc, permute_sc


_noise_cache = []


def _noise_pad_const():
    if not _noise_cache:
        with jax.ensure_compile_time_eval():
            noise = (jax.random.uniform(jax.random.key(1), (1, _N),
                                        dtype=jnp.float32) * 0.1)
            arr = np.asarray(noise[0])
        _noise_cache.append(np.concatenate(
            [arr, np.full(_NP - _N, np.inf, np.float32)]))
    return _noise_cache[0]


def kernel(x, edge_index):
    hist_sc, permute_sc = _sc_kernels()
    x2 = x if x.ndim == 2 else x[0]
    edges = edge_index.astype(jnp.int32).reshape(2, _ER, 128)
    noise_pad = jnp.asarray(_noise_pad_const())

    hist3 = hist_sc(edges)
    noise_r = noise_pad.reshape(_NP // _JC, _JC)
    histT = hist3.reshape(2, _NP).T
    noise_c = noise_pad.reshape(_NP, 1)

    rowc, colc = _rank_tc(hist3, noise_r, histT, noise_c)
    rowf = rowc.reshape(_NP)
    colf = colc.reshape(_NP)

    xs, sidx = permute_sc(x2, rowf, colf)
    return (xs[None], sidx[None])
